# baseline (speedup 1.0000x reference)
import functools

import jax
import jax.numpy as jnp
from jax import lax
from jax.experimental import pallas as pl
from jax.experimental.pallas import tpu as pltpu
from jax.experimental.pallas import tpu_sc as plsc

L = 16
NC = 2
NS = 16
NW = NC * NS
H = 500224


def _tc_fuse_table(tblT):
    BR = 512
    G = H // BR

    def body(a_ref, b_ref, o_ref):
        o_ref[...] = jnp.concatenate(
            [a_ref[...].T, b_ref[...].T], axis=1)

    return pl.pallas_call(
        body,
        grid=(G,),
        in_specs=[pl.BlockSpec((64, BR), lambda i: (0, i)),
                  pl.BlockSpec((64, BR), lambda i: (0, i + G))],
        out_specs=pl.BlockSpec((BR, 128), lambda i: (i, 0)),
        out_shape=jax.ShapeDtypeStruct((H, 128), jnp.float32),
    )(tblT, tblT)


def _sc_sims(tbl2, u2, uo, v2, vo, n2, no, B, K, D):
    W = 2 * D
    b_per_w = B // NW
    NBLK = 64
    nblocks = b_per_w // NBLK
    nchunk = NBLK * K // 128

    mesh = plsc.VectorSubcoreMesh(core_axis_name="c", subcore_axis_name="s")

    @functools.partial(
        pl.kernel,
        mesh=mesh,
        compiler_params=pltpu.CompilerParams(
            needs_layout_passes=False, use_tc_tiling_on_sc=False),
        out_type=jax.ShapeDtypeStruct(((K + 1) * B,), jnp.float32),
        scratch_types=[
            pltpu.VMEM((NBLK,), jnp.int32),
            pltpu.VMEM((NBLK + L,), jnp.int32),
            pltpu.VMEM((NBLK,), jnp.int32),
            pltpu.VMEM((NBLK + L,), jnp.int32),
            pltpu.VMEM((NBLK * K,), jnp.int32),
            pltpu.VMEM((NBLK * K + L,), jnp.int32),
            pltpu.VMEM((NBLK, W), jnp.float32),
            pltpu.VMEM((NBLK, W), jnp.float32),
            pltpu.VMEM((NBLK * K, W), jnp.float32),
            pltpu.VMEM(((K + 1) * b_per_w,), jnp.float32),
            pltpu.SemaphoreType.DMA,
        ],
    )
    def sc_kernel(tbl_h, u2_h, uo_h, v2_h, vo_h, n2_h, no_h, sims_h,
                  u_idx, u_off, v_idx, v_off, neg_idx, neg_off,
                  u_rows, v_rows, neg_rows, sims_v, sem):
        wid = lax.axis_index("s") * NC + lax.axis_index("c")
        base = wid * b_per_w
        iota = lax.iota(jnp.int32, L)
        for blk in range(nblocks):
            boff = blk * NBLK
            pltpu.sync_copy(u2_h.at[pl.ds(base + boff, NBLK)], u_idx)
            pltpu.sync_copy(uo_h.at[pl.ds(base + boff, NBLK)],
                            u_off.at[pl.ds(0, NBLK)])
            pltpu.sync_copy(v2_h.at[pl.ds(base + boff, NBLK)], v_idx)
            pltpu.sync_copy(vo_h.at[pl.ds(base + boff, NBLK)],
                            v_off.at[pl.ds(0, NBLK)])
            pltpu.sync_copy(
                n2_h.at[pl.ds((base + boff) * K, NBLK * K)], neg_idx)
            pltpu.sync_copy(
                no_h.at[pl.ds((base + boff) * K, NBLK * K)],
                neg_off.at[pl.ds(0, NBLK * K)])
            cps = [pltpu.async_copy(tbl_h.at[u_idx], u_rows, sem),
                   pltpu.async_copy(tbl_h.at[v_idx], v_rows, sem)]
            for j in range(nchunk):
                cps.append(pltpu.async_copy(
                    tbl_h.at[neg_idx.at[pl.ds(j * 128, 128)]],
                    neg_rows.at[pl.ds(j * 128, 128), :], sem))
            for c in cps:
                c.wait()
            nch = D // L

            def group(g, _, boff=boff):
                def body(lb, accs):
                    row = g * L + lb
                    ov = v_off[pl.ds(row, L)][0]
                    ou = u_off[pl.ds(row, L)][0]
                    non = neg_off[pl.ds(row * K, L)]
                    vvec = [v_rows[row, pl.ds(ov + c * L, L)]
                            for c in range(nch)]
                    uvec = [u_rows[row, pl.ds(ou + c * L, L)]
                            for c in range(nch)]
                    s = uvec[0] * vvec[0]
                    for c in range(1, nch):
                        s = s + uvec[c] * vvec[c]
                    lane = iota == lb
                    out = [jnp.where(lane, jnp.sum(s), accs[0])]
                    for k in range(K):
                        nr = row * K + k
                        on = non[k]
                        s = vvec[0] * neg_rows[nr, pl.ds(on, L)]
                        for c in range(1, nch):
                            s = s + vvec[c] * neg_rows[nr, pl.ds(on + c * L, L)]
                        out.append(jnp.where(lane, jnp.sum(s), accs[k + 1]))
                    return tuple(out)
                accs = lax.fori_loop(
                    0, L, body,
                    tuple(jnp.zeros((L,), jnp.float32) for _ in range(K + 1)))
                gb = boff + g * L
                sims_v[pl.ds(gb, L)] = -accs[0]
                for k in range(K):
                    sims_v[pl.ds((1 + k) * b_per_w + gb, L)] = accs[k + 1]
                return 0

            lax.fori_loop(0, NBLK // L, group, 0)
        for j in range(K + 1):
            pltpu.sync_copy(sims_v.at[pl.ds(j * b_per_w, b_per_w)],
                            sims_h.at[pl.ds(j * B + base, b_per_w)])

    return sc_kernel(tbl2, u2, uo, v2, vo, n2, no)


def _tc_logsigmoid_sum(sims, B, K):
    BT = 2048

    def body(s_ref, o_ref):
        x = s_ref[...]
        sp = jnp.maximum(x, 0.0) + jnp.log1p(jnp.exp(-jnp.abs(x)))
        o_ref[...] = jnp.sum(sp, axis=0, keepdims=True)

    return pl.pallas_call(
        body,
        grid=(B // BT,),
        in_specs=[pl.BlockSpec((K + 1, BT), lambda i: (0, i))],
        out_specs=pl.BlockSpec((1, BT), lambda i: (0, i)),
        out_shape=jax.ShapeDtypeStruct((1, B), jnp.float32),
    )(sims)


def kernel(u, v, neg, table):
    B = u.shape[0]
    K = neg.shape[1]
    V, D = table.shape
    tbl2 = _tc_fuse_table(table.T)
    u32 = u.astype(jnp.int32)
    v32 = v.astype(jnp.int32)
    ng = neg.astype(jnp.int32).reshape(-1)
    u2 = jnp.where(u32 < H, u32, u32 - H)
    uo = jnp.where(u32 < H, 0, D)
    v2 = jnp.where(v32 < H, v32, v32 - H)
    vo = jnp.where(v32 < H, 0, D)
    n2 = jnp.where(ng < H, ng, ng - H)
    no = jnp.where(ng < H, 0, D)
    sims_flat = _sc_sims(tbl2, u2, uo, v2, vo, n2, no, B, K, D)
    sims = sims_flat.reshape(K + 1, B)
    p = _tc_logsigmoid_sum(sims, B, K)
    return p.reshape(B)


# speedup vs baseline: 2.0879x; 2.0879x over previous
import functools

import jax
import jax.numpy as jnp
from jax import lax
from jax.experimental import pallas as pl
from jax.experimental.pallas import tpu as pltpu
from jax.experimental.pallas import tpu_sc as plsc

L = 16
NC = 2
NS = 16
NW = NC * NS
H = 524288


def _tc_fuse_table(tblT):
    BR = 4096
    G = H // BR
    V = 1000000
    last = (V - 1) // BR

    def body(a_ref, b_ref, o_ref):
        o_ref[...] = jnp.concatenate(
            [a_ref[...].T, b_ref[...].T], axis=1)

    return pl.pallas_call(
        body,
        grid=(G,),
        in_specs=[pl.BlockSpec((64, BR), lambda i: (0, i)),
                  pl.BlockSpec((64, BR),
                               lambda i: (0, jnp.minimum(i + G, last)))],
        out_specs=pl.BlockSpec((BR, 128), lambda i: (i, 0)),
        out_shape=jax.ShapeDtypeStruct((H, 128), jnp.float32),
    )(tblT, tblT)


def _sc_sims(tbl2, u2, uo, v2, vo, n2, no, B, K, D):
    W = 2 * D
    b_per_w = B // NW
    NBLK = 64
    nblocks = b_per_w // NBLK
    nchunk = NBLK * K // 128

    mesh = plsc.VectorSubcoreMesh(core_axis_name="c", subcore_axis_name="s")

    @functools.partial(
        pl.kernel,
        mesh=mesh,
        compiler_params=pltpu.CompilerParams(
            needs_layout_passes=False, use_tc_tiling_on_sc=False),
        out_type=jax.ShapeDtypeStruct(((K + 1) * B,), jnp.float32),
        scratch_types=[
            pltpu.VMEM((NBLK,), jnp.int32),
            pltpu.VMEM((NBLK + L,), jnp.int32),
            pltpu.VMEM((NBLK,), jnp.int32),
            pltpu.VMEM((NBLK + L,), jnp.int32),
            pltpu.VMEM((NBLK * K,), jnp.int32),
            pltpu.VMEM((NBLK * K + L,), jnp.int32),
            pltpu.VMEM((NBLK, W), jnp.float32),
            pltpu.VMEM((NBLK, W), jnp.float32),
            pltpu.VMEM((NBLK * K, W), jnp.float32),
            pltpu.VMEM(((K + 1) * b_per_w,), jnp.float32),
            pltpu.SemaphoreType.DMA,
        ],
    )
    def sc_kernel(tbl_h, u2_h, uo_h, v2_h, vo_h, n2_h, no_h, sims_h,
                  u_idx, u_off, v_idx, v_off, neg_idx, neg_off,
                  u_rows, v_rows, neg_rows, sims_v, sem):
        wid = lax.axis_index("s") * NC + lax.axis_index("c")
        base = wid * b_per_w
        iota = lax.iota(jnp.int32, L)
        for blk in range(nblocks):
            boff = blk * NBLK
            pltpu.sync_copy(u2_h.at[pl.ds(base + boff, NBLK)], u_idx)
            pltpu.sync_copy(uo_h.at[pl.ds(base + boff, NBLK)],
                            u_off.at[pl.ds(0, NBLK)])
            pltpu.sync_copy(v2_h.at[pl.ds(base + boff, NBLK)], v_idx)
            pltpu.sync_copy(vo_h.at[pl.ds(base + boff, NBLK)],
                            v_off.at[pl.ds(0, NBLK)])
            pltpu.sync_copy(
                n2_h.at[pl.ds((base + boff) * K, NBLK * K)], neg_idx)
            pltpu.sync_copy(
                no_h.at[pl.ds((base + boff) * K, NBLK * K)],
                neg_off.at[pl.ds(0, NBLK * K)])
            cps = [pltpu.async_copy(tbl_h.at[u_idx], u_rows, sem),
                   pltpu.async_copy(tbl_h.at[v_idx], v_rows, sem)]
            for j in range(nchunk):
                cps.append(pltpu.async_copy(
                    tbl_h.at[neg_idx.at[pl.ds(j * 128, 128)]],
                    neg_rows.at[pl.ds(j * 128, 128), :], sem))
            for c in cps:
                c.wait()
            nch = D // L

            def group(g, _, boff=boff):
                def body(lb, accs):
                    row = g * L + lb
                    ov = v_off[pl.ds(row, L)][0]
                    ou = u_off[pl.ds(row, L)][0]
                    non = neg_off[pl.ds(row * K, L)]
                    vvec = [v_rows[row, pl.ds(ov + c * L, L)]
                            for c in range(nch)]
                    uvec = [u_rows[row, pl.ds(ou + c * L, L)]
                            for c in range(nch)]
                    s = uvec[0] * vvec[0]
                    for c in range(1, nch):
                        s = s + uvec[c] * vvec[c]
                    lane = iota == lb
                    out = [jnp.where(lane, jnp.sum(s), accs[0])]
                    for k in range(K):
                        nr = row * K + k
                        on = non[k]
                        s = vvec[0] * neg_rows[nr, pl.ds(on, L)]
                        for c in range(1, nch):
                            s = s + vvec[c] * neg_rows[nr, pl.ds(on + c * L, L)]
                        out.append(jnp.where(lane, jnp.sum(s), accs[k + 1]))
                    return tuple(out)
                accs = lax.fori_loop(
                    0, L, body,
                    tuple(jnp.zeros((L,), jnp.float32) for _ in range(K + 1)))
                gb = boff + g * L
                sims_v[pl.ds(gb, L)] = -accs[0]
                for k in range(K):
                    sims_v[pl.ds((1 + k) * b_per_w + gb, L)] = accs[k + 1]
                return 0

            lax.fori_loop(0, NBLK // L, group, 0)
        for j in range(K + 1):
            pltpu.sync_copy(sims_v.at[pl.ds(j * b_per_w, b_per_w)],
                            sims_h.at[pl.ds(j * B + base, b_per_w)])

    return sc_kernel(tbl2, u2, uo, v2, vo, n2, no)


def _tc_logsigmoid_sum(sims, B, K):
    BT = 2048

    def body(s_ref, o_ref):
        x = s_ref[...]
        sp = jnp.maximum(x, 0.0) + jnp.log1p(jnp.exp(-jnp.abs(x)))
        o_ref[...] = jnp.sum(sp, axis=0, keepdims=True)

    return pl.pallas_call(
        body,
        grid=(B // BT,),
        in_specs=[pl.BlockSpec((K + 1, BT), lambda i: (0, i))],
        out_specs=pl.BlockSpec((1, BT), lambda i: (0, i)),
        out_shape=jax.ShapeDtypeStruct((1, B), jnp.float32),
    )(sims)


def kernel(u, v, neg, table):
    B = u.shape[0]
    K = neg.shape[1]
    V, D = table.shape
    tbl2 = _tc_fuse_table(table.T)
    u32 = u.astype(jnp.int32)
    v32 = v.astype(jnp.int32)
    ng = neg.astype(jnp.int32).reshape(-1)
    u2 = jnp.where(u32 < H, u32, u32 - H)
    uo = jnp.where(u32 < H, 0, D)
    v2 = jnp.where(v32 < H, v32, v32 - H)
    vo = jnp.where(v32 < H, 0, D)
    n2 = jnp.where(ng < H, ng, ng - H)
    no = jnp.where(ng < H, 0, D)
    sims_flat = _sc_sims(tbl2, u2, uo, v2, vo, n2, no, B, K, D)
    sims = sims_flat.reshape(K + 1, B)
    p = _tc_logsigmoid_sum(sims, B, K)
    return p.reshape(B)


# speedup vs baseline: 2.2933x; 1.0984x over previous
import functools

import jax
import jax.numpy as jnp
from jax import lax
from jax.experimental import pallas as pl
from jax.experimental.pallas import tpu as pltpu
from jax.experimental.pallas import tpu_sc as plsc

L = 16
NC = 2
NS = 16
NW = NC * NS
H = 524288


def _tc_fuse_table(tblT):
    BR = 8192
    G = H // BR
    V = 1000000
    last = (V - 1) // BR

    def body(a_ref, b_ref, o_ref):
        o_ref[...] = jnp.concatenate(
            [a_ref[...].T, b_ref[...].T], axis=1)

    return pl.pallas_call(
        body,
        grid=(G,),
        in_specs=[pl.BlockSpec((64, BR), lambda i: (0, i)),
                  pl.BlockSpec((64, BR),
                               lambda i: (0, jnp.minimum(i + G, last)))],
        out_specs=pl.BlockSpec((BR, 128), lambda i: (i, 0)),
        out_shape=jax.ShapeDtypeStruct((H, 128), jnp.float32),
    )(tblT, tblT)


def _sc_sims(tbl2, u2, uo, v2, vo, n2, no, B, K, D):
    W = 2 * D
    b_per_w = B // NW
    NBLK = 64
    nblocks = b_per_w // NBLK
    nchunk = NBLK * K // 128

    mesh = plsc.VectorSubcoreMesh(core_axis_name="c", subcore_axis_name="s")

    @functools.partial(
        pl.kernel,
        mesh=mesh,
        compiler_params=pltpu.CompilerParams(
            needs_layout_passes=False, use_tc_tiling_on_sc=False),
        out_type=jax.ShapeDtypeStruct(((K + 1) * B,), jnp.float32),
        scratch_types=[
            pltpu.VMEM((NBLK,), jnp.int32),
            pltpu.VMEM((NBLK + L,), jnp.int32),
            pltpu.VMEM((NBLK,), jnp.int32),
            pltpu.VMEM((NBLK + L,), jnp.int32),
            pltpu.VMEM((NBLK * K,), jnp.int32),
            pltpu.VMEM((NBLK * K + L,), jnp.int32),
            pltpu.VMEM((NBLK, W), jnp.float32),
            pltpu.VMEM((NBLK, W), jnp.float32),
            pltpu.VMEM((NBLK * K, W), jnp.float32),
            pltpu.VMEM(((K + 1) * b_per_w,), jnp.float32),
            pltpu.SemaphoreType.DMA,
        ],
    )
    def sc_kernel(tbl_h, u2_h, uo_h, v2_h, vo_h, n2_h, no_h, sims_h,
                  u_idx, u_off, v_idx, v_off, neg_idx, neg_off,
                  u_rows, v_rows, neg_rows, sims_v, sem):
        wid = lax.axis_index("s") * NC + lax.axis_index("c")
        base = wid * b_per_w
        iota = lax.iota(jnp.int32, L)
        for blk in range(nblocks):
            boff = blk * NBLK
            pltpu.sync_copy(u2_h.at[pl.ds(base + boff, NBLK)], u_idx)
            pltpu.sync_copy(uo_h.at[pl.ds(base + boff, NBLK)],
                            u_off.at[pl.ds(0, NBLK)])
            pltpu.sync_copy(v2_h.at[pl.ds(base + boff, NBLK)], v_idx)
            pltpu.sync_copy(vo_h.at[pl.ds(base + boff, NBLK)],
                            v_off.at[pl.ds(0, NBLK)])
            pltpu.sync_copy(
                n2_h.at[pl.ds((base + boff) * K, NBLK * K)], neg_idx)
            pltpu.sync_copy(
                no_h.at[pl.ds((base + boff) * K, NBLK * K)],
                neg_off.at[pl.ds(0, NBLK * K)])
            cps = [pltpu.async_copy(tbl_h.at[u_idx], u_rows, sem),
                   pltpu.async_copy(tbl_h.at[v_idx], v_rows, sem)]
            for j in range(nchunk):
                cps.append(pltpu.async_copy(
                    tbl_h.at[neg_idx.at[pl.ds(j * 128, 128)]],
                    neg_rows.at[pl.ds(j * 128, 128), :], sem))
            for c in cps:
                c.wait()
            nch = D // L

            def group(g, _, boff=boff):
                def body(lb, accs):
                    row = g * L + lb
                    ov = v_off[pl.ds(row, L)][0]
                    ou = u_off[pl.ds(row, L)][0]
                    non = neg_off[pl.ds(row * K, L)]
                    vvec = [v_rows[row, pl.ds(ov + c * L, L)]
                            for c in range(nch)]
                    uvec = [u_rows[row, pl.ds(ou + c * L, L)]
                            for c in range(nch)]
                    s = uvec[0] * vvec[0]
                    for c in range(1, nch):
                        s = s + uvec[c] * vvec[c]
                    lane = iota == lb
                    out = [jnp.where(lane, jnp.sum(s), accs[0])]
                    for k in range(K):
                        nr = row * K + k
                        on = non[k]
                        s = vvec[0] * neg_rows[nr, pl.ds(on, L)]
                        for c in range(1, nch):
                            s = s + vvec[c] * neg_rows[nr, pl.ds(on + c * L, L)]
                        out.append(jnp.where(lane, jnp.sum(s), accs[k + 1]))
                    return tuple(out)
                accs = lax.fori_loop(
                    0, L, body,
                    tuple(jnp.zeros((L,), jnp.float32) for _ in range(K + 1)))
                gb = boff + g * L
                sims_v[pl.ds(gb, L)] = -accs[0]
                for k in range(K):
                    sims_v[pl.ds((1 + k) * b_per_w + gb, L)] = accs[k + 1]
                return 0

            lax.fori_loop(0, NBLK // L, group, 0)
        for j in range(K + 1):
            pltpu.sync_copy(sims_v.at[pl.ds(j * b_per_w, b_per_w)],
                            sims_h.at[pl.ds(j * B + base, b_per_w)])

    return sc_kernel(tbl2, u2, uo, v2, vo, n2, no)


def _tc_logsigmoid_sum(sims, B, K):
    BT = 2048

    def body(s_ref, o_ref):
        x = s_ref[...]
        sp = jnp.maximum(x, 0.0) + jnp.log1p(jnp.exp(-jnp.abs(x)))
        o_ref[...] = jnp.sum(sp, axis=0, keepdims=True)

    return pl.pallas_call(
        body,
        grid=(B // BT,),
        in_specs=[pl.BlockSpec((K + 1, BT), lambda i: (0, i))],
        out_specs=pl.BlockSpec((1, BT), lambda i: (0, i)),
        out_shape=jax.ShapeDtypeStruct((1, B), jnp.float32),
    )(sims)


def kernel(u, v, neg, table):
    B = u.shape[0]
    K = neg.shape[1]
    V, D = table.shape
    tbl2 = _tc_fuse_table(table.T)
    u32 = u.astype(jnp.int32)
    v32 = v.astype(jnp.int32)
    ng = neg.astype(jnp.int32).reshape(-1)
    u2 = jnp.where(u32 < H, u32, u32 - H)
    uo = jnp.where(u32 < H, 0, D)
    v2 = jnp.where(v32 < H, v32, v32 - H)
    vo = jnp.where(v32 < H, 0, D)
    n2 = jnp.where(ng < H, ng, ng - H)
    no = jnp.where(ng < H, 0, D)
    sims_flat = _sc_sims(tbl2, u2, uo, v2, vo, n2, no, B, K, D)
    sims = sims_flat.reshape(K + 1, B)
    p = _tc_logsigmoid_sum(sims, B, K)
    return p.reshape(B)


# speedup vs baseline: 2.3863x; 1.0406x over previous
import functools

import jax
import jax.numpy as jnp
from jax import lax
from jax.experimental import pallas as pl
from jax.experimental.pallas import tpu as pltpu
from jax.experimental.pallas import tpu_sc as plsc

L = 16
NC = 2
NS = 16
NW = NC * NS
H = 524288


def _tc_fuse_table(tblT):
    BR = 16384
    G = H // BR
    V = 1000000
    last = (V - 1) // BR

    def body(a_ref, b_ref, o_ref):
        o_ref[...] = jnp.concatenate(
            [a_ref[...].T, b_ref[...].T], axis=1)

    return pl.pallas_call(
        body,
        grid=(G,),
        in_specs=[pl.BlockSpec((64, BR), lambda i: (0, i)),
                  pl.BlockSpec((64, BR),
                               lambda i: (0, jnp.minimum(i + G, last)))],
        out_specs=pl.BlockSpec((BR, 128), lambda i: (i, 0)),
        out_shape=jax.ShapeDtypeStruct((H, 128), jnp.float32),
    )(tblT, tblT)


def _sc_sims(tbl2, u2, uo, v2, vo, n2, no, B, K, D):
    W = 2 * D
    b_per_w = B // NW
    NBLK = 64
    nblocks = b_per_w // NBLK
    nchunk = NBLK * K // 128

    mesh = plsc.VectorSubcoreMesh(core_axis_name="c", subcore_axis_name="s")

    @functools.partial(
        pl.kernel,
        mesh=mesh,
        compiler_params=pltpu.CompilerParams(
            needs_layout_passes=False, use_tc_tiling_on_sc=False),
        out_type=jax.ShapeDtypeStruct(((K + 1) * B,), jnp.float32),
        scratch_types=[
            pltpu.VMEM((NBLK,), jnp.int32),
            pltpu.VMEM((NBLK + L,), jnp.int32),
            pltpu.VMEM((NBLK,), jnp.int32),
            pltpu.VMEM((NBLK + L,), jnp.int32),
            pltpu.VMEM((NBLK * K,), jnp.int32),
            pltpu.VMEM((NBLK * K + L,), jnp.int32),
            pltpu.VMEM((NBLK, W), jnp.float32),
            pltpu.VMEM((NBLK, W), jnp.float32),
            pltpu.VMEM((NBLK * K, W), jnp.float32),
            pltpu.VMEM(((K + 1) * b_per_w,), jnp.float32),
            pltpu.SemaphoreType.DMA,
        ],
    )
    def sc_kernel(tbl_h, u2_h, uo_h, v2_h, vo_h, n2_h, no_h, sims_h,
                  u_idx, u_off, v_idx, v_off, neg_idx, neg_off,
                  u_rows, v_rows, neg_rows, sims_v, sem):
        wid = lax.axis_index("s") * NC + lax.axis_index("c")
        base = wid * b_per_w
        iota = lax.iota(jnp.int32, L)
        for blk in range(nblocks):
            boff = blk * NBLK
            pltpu.sync_copy(u2_h.at[pl.ds(base + boff, NBLK)], u_idx)
            pltpu.sync_copy(uo_h.at[pl.ds(base + boff, NBLK)],
                            u_off.at[pl.ds(0, NBLK)])
            pltpu.sync_copy(v2_h.at[pl.ds(base + boff, NBLK)], v_idx)
            pltpu.sync_copy(vo_h.at[pl.ds(base + boff, NBLK)],
                            v_off.at[pl.ds(0, NBLK)])
            pltpu.sync_copy(
                n2_h.at[pl.ds((base + boff) * K, NBLK * K)], neg_idx)
            pltpu.sync_copy(
                no_h.at[pl.ds((base + boff) * K, NBLK * K)],
                neg_off.at[pl.ds(0, NBLK * K)])
            cps = [pltpu.async_copy(tbl_h.at[u_idx], u_rows, sem),
                   pltpu.async_copy(tbl_h.at[v_idx], v_rows, sem)]
            for j in range(nchunk):
                cps.append(pltpu.async_copy(
                    tbl_h.at[neg_idx.at[pl.ds(j * 128, 128)]],
                    neg_rows.at[pl.ds(j * 128, 128), :], sem))
            for c in cps:
                c.wait()
            nch = D // L

            def group(g, _, boff=boff):
                def body(lb, accs):
                    row = g * L + lb
                    ov = v_off[pl.ds(row, L)][0]
                    ou = u_off[pl.ds(row, L)][0]
                    non = neg_off[pl.ds(row * K, L)]
                    vvec = [v_rows[row, pl.ds(ov + c * L, L)]
                            for c in range(nch)]
                    uvec = [u_rows[row, pl.ds(ou + c * L, L)]
                            for c in range(nch)]
                    s = uvec[0] * vvec[0]
                    for c in range(1, nch):
                        s = s + uvec[c] * vvec[c]
                    lane = iota == lb
                    out = [jnp.where(lane, jnp.sum(s), accs[0])]
                    for k in range(K):
                        nr = row * K + k
                        on = non[k]
                        s = vvec[0] * neg_rows[nr, pl.ds(on, L)]
                        for c in range(1, nch):
                            s = s + vvec[c] * neg_rows[nr, pl.ds(on + c * L, L)]
                        out.append(jnp.where(lane, jnp.sum(s), accs[k + 1]))
                    return tuple(out)
                accs = lax.fori_loop(
                    0, L, body,
                    tuple(jnp.zeros((L,), jnp.float32) for _ in range(K + 1)))
                gb = boff + g * L
                sims_v[pl.ds(gb, L)] = -accs[0]
                for k in range(K):
                    sims_v[pl.ds((1 + k) * b_per_w + gb, L)] = accs[k + 1]
                return 0

            lax.fori_loop(0, NBLK // L, group, 0)
        for j in range(K + 1):
            pltpu.sync_copy(sims_v.at[pl.ds(j * b_per_w, b_per_w)],
                            sims_h.at[pl.ds(j * B + base, b_per_w)])

    return sc_kernel(tbl2, u2, uo, v2, vo, n2, no)


def _tc_logsigmoid_sum(sims, B, K):
    BT = 2048

    def body(s_ref, o_ref):
        x = s_ref[...]
        sp = jnp.maximum(x, 0.0) + jnp.log1p(jnp.exp(-jnp.abs(x)))
        o_ref[...] = jnp.sum(sp, axis=0, keepdims=True)

    return pl.pallas_call(
        body,
        grid=(B // BT,),
        in_specs=[pl.BlockSpec((K + 1, BT), lambda i: (0, i))],
        out_specs=pl.BlockSpec((1, BT), lambda i: (0, i)),
        out_shape=jax.ShapeDtypeStruct((1, B), jnp.float32),
    )(sims)


def kernel(u, v, neg, table):
    B = u.shape[0]
    K = neg.shape[1]
    V, D = table.shape
    tbl2 = _tc_fuse_table(table.T)
    u32 = u.astype(jnp.int32)
    v32 = v.astype(jnp.int32)
    ng = neg.astype(jnp.int32).reshape(-1)
    u2 = jnp.where(u32 < H, u32, u32 - H)
    uo = jnp.where(u32 < H, 0, D)
    v2 = jnp.where(v32 < H, v32, v32 - H)
    vo = jnp.where(v32 < H, 0, D)
    n2 = jnp.where(ng < H, ng, ng - H)
    no = jnp.where(ng < H, 0, D)
    sims_flat = _sc_sims(tbl2, u2, uo, v2, vo, n2, no, B, K, D)
    sims = sims_flat.reshape(K + 1, B)
    p = _tc_logsigmoid_sum(sims, B, K)
    return p.reshape(B)


# speedup vs baseline: 2.4559x; 1.0292x over previous
import functools

import jax
import jax.numpy as jnp
from jax import lax
from jax.experimental import pallas as pl
from jax.experimental.pallas import tpu as pltpu
from jax.experimental.pallas import tpu_sc as plsc

L = 16
NC = 2
NS = 16
NW = NC * NS
H = 524288


def _tc_fuse_table(tblT):
    BR = 16384
    G = H // BR
    V = 1000000
    last = (V - 1) // BR

    def body(a_ref, b_ref, o_ref):
        o_ref[...] = jnp.concatenate(
            [a_ref[...].T, b_ref[...].T], axis=1)

    return pl.pallas_call(
        body,
        grid=(G,),
        in_specs=[pl.BlockSpec((64, BR), lambda i: (0, i)),
                  pl.BlockSpec((64, BR),
                               lambda i: (0, jnp.minimum(i + G, last)))],
        out_specs=pl.BlockSpec((BR, 128), lambda i: (i, 0)),
        out_shape=jax.ShapeDtypeStruct((H, 128), jnp.float32),
    )(tblT, tblT)


def _sc_sims(tbl2, u2, uo, v2, vo, n2, no, B, K, D):
    W = 2 * D
    b_per_w = B // NW
    NBLK = 32
    nblocks = b_per_w // NBLK
    CH = 64
    nchunk = NBLK * K // CH

    mesh = plsc.VectorSubcoreMesh(core_axis_name="c", subcore_axis_name="s")

    def _bufset():
        return [
            pltpu.VMEM((NBLK,), jnp.int32),
            pltpu.VMEM((NBLK + L,), jnp.int32),
            pltpu.VMEM((NBLK,), jnp.int32),
            pltpu.VMEM((NBLK + L,), jnp.int32),
            pltpu.VMEM((NBLK * K,), jnp.int32),
            pltpu.VMEM((NBLK * K + L,), jnp.int32),
            pltpu.VMEM((NBLK, W), jnp.float32),
            pltpu.VMEM((NBLK, W), jnp.float32),
            pltpu.VMEM((NBLK * K, W), jnp.float32),
        ]

    @functools.partial(
        pl.kernel,
        mesh=mesh,
        compiler_params=pltpu.CompilerParams(
            needs_layout_passes=False, use_tc_tiling_on_sc=False),
        out_type=jax.ShapeDtypeStruct(((K + 1) * B,), jnp.float32),
        scratch_types=_bufset() + _bufset() + [
            pltpu.VMEM(((K + 1) * b_per_w,), jnp.float32),
            pltpu.SemaphoreType.DMA,
            pltpu.SemaphoreType.DMA,
        ],
    )
    def sc_kernel(tbl_h, u2_h, uo_h, v2_h, vo_h, n2_h, no_h, sims_h, *sc):
        sets = [(sc[0:9], sc[19]), (sc[9:18], sc[20])]
        sims_v = sc[18]
        wid = lax.axis_index("s") * NC + lax.axis_index("c")
        base = wid * b_per_w
        iota = lax.iota(jnp.int32, L)
        nch = D // L

        def stage(blk, st, sem):
            (u_idx, u_off, v_idx, v_off, neg_idx, neg_off,
             u_rows, v_rows, neg_rows) = st
            boff = blk * NBLK
            pltpu.sync_copy(u2_h.at[pl.ds(base + boff, NBLK)], u_idx)
            pltpu.sync_copy(uo_h.at[pl.ds(base + boff, NBLK)],
                            u_off.at[pl.ds(0, NBLK)])
            pltpu.sync_copy(v2_h.at[pl.ds(base + boff, NBLK)], v_idx)
            pltpu.sync_copy(vo_h.at[pl.ds(base + boff, NBLK)],
                            v_off.at[pl.ds(0, NBLK)])
            pltpu.sync_copy(
                n2_h.at[pl.ds((base + boff) * K, NBLK * K)], neg_idx)
            pltpu.sync_copy(
                no_h.at[pl.ds((base + boff) * K, NBLK * K)],
                neg_off.at[pl.ds(0, NBLK * K)])
            cps = [pltpu.async_copy(tbl_h.at[u_idx], u_rows, sem),
                   pltpu.async_copy(tbl_h.at[v_idx], v_rows, sem)]
            for j in range(nchunk):
                cps.append(pltpu.async_copy(
                    tbl_h.at[neg_idx.at[pl.ds(j * CH, CH)]],
                    neg_rows.at[pl.ds(j * CH, CH), :], sem))
            return cps

        def compute(blk, st):
            (u_idx, u_off, v_idx, v_off, neg_idx, neg_off,
             u_rows, v_rows, neg_rows) = st
            boff = blk * NBLK

            def group(g, _):
                def body(lb, accs):
                    row = g * L + lb
                    ov = v_off[pl.ds(row, L)][0]
                    ou = u_off[pl.ds(row, L)][0]
                    non = neg_off[pl.ds(row * K, L)]
                    vvec = [v_rows[row, pl.ds(ov + c * L, L)]
                            for c in range(nch)]
                    uvec = [u_rows[row, pl.ds(ou + c * L, L)]
                            for c in range(nch)]
                    s = uvec[0] * vvec[0]
                    for c in range(1, nch):
                        s = s + uvec[c] * vvec[c]
                    lane = iota == lb
                    out = [jnp.where(lane, jnp.sum(s), accs[0])]
                    for k in range(K):
                        nr = row * K + k
                        on = non[k]
                        s = vvec[0] * neg_rows[nr, pl.ds(on, L)]
                        for c in range(1, nch):
                            s = s + vvec[c] * neg_rows[nr, pl.ds(on + c * L, L)]
                        out.append(jnp.where(lane, jnp.sum(s), accs[k + 1]))
                    return tuple(out)
                accs = lax.fori_loop(
                    0, L, body,
                    tuple(jnp.zeros((L,), jnp.float32) for _ in range(K + 1)))
                gb = boff + g * L
                sims_v[pl.ds(gb, L)] = -accs[0]
                for k in range(K):
                    sims_v[pl.ds((1 + k) * b_per_w + gb, L)] = accs[k + 1]
                return 0

            lax.fori_loop(0, NBLK // L, group, 0)

        pend = stage(0, *sets[0])
        for blk in range(nblocks):
            st, _ = sets[blk % 2]
            cur = pend
            if blk + 1 < nblocks:
                pend = stage(blk + 1, *sets[(blk + 1) % 2])
            for c in cur:
                c.wait()
            compute(blk, st)
        for j in range(K + 1):
            pltpu.sync_copy(sims_v.at[pl.ds(j * b_per_w, b_per_w)],
                            sims_h.at[pl.ds(j * B + base, b_per_w)])

    return sc_kernel(tbl2, u2, uo, v2, vo, n2, no)


def _tc_logsigmoid_sum(sims, B, K):
    BT = 2048

    def body(s_ref, o_ref):
        x = s_ref[...]
        sp = jnp.maximum(x, 0.0) + jnp.log1p(jnp.exp(-jnp.abs(x)))
        o_ref[...] = jnp.sum(sp, axis=0, keepdims=True)

    return pl.pallas_call(
        body,
        grid=(B // BT,),
        in_specs=[pl.BlockSpec((K + 1, BT), lambda i: (0, i))],
        out_specs=pl.BlockSpec((1, BT), lambda i: (0, i)),
        out_shape=jax.ShapeDtypeStruct((1, B), jnp.float32),
    )(sims)


def kernel(u, v, neg, table):
    B = u.shape[0]
    K = neg.shape[1]
    V, D = table.shape
    tbl2 = _tc_fuse_table(table.T)
    u32 = u.astype(jnp.int32)
    v32 = v.astype(jnp.int32)
    ng = neg.astype(jnp.int32).reshape(-1)
    u2 = jnp.where(u32 < H, u32, u32 - H)
    uo = jnp.where(u32 < H, 0, D)
    v2 = jnp.where(v32 < H, v32, v32 - H)
    vo = jnp.where(v32 < H, 0, D)
    n2 = jnp.where(ng < H, ng, ng - H)
    no = jnp.where(ng < H, 0, D)
    sims_flat = _sc_sims(tbl2, u2, uo, v2, vo, n2, no, B, K, D)
    sims = sims_flat.reshape(K + 1, B)
    p = _tc_logsigmoid_sum(sims, B, K)
    return p.reshape(B)
